# baseline (speedup 1.0000x reference)
import functools

import jax
import jax.numpy as jnp
from jax import lax
from jax.experimental import pallas as pl
from jax.experimental.pallas import tpu as pltpu
from jax.experimental.pallas import tpu_sc as plsc

EMB_ = 1024
HID_ = 2048
NEXP_ = 8
NTOK_ = 4096
CHUNK_ = 512
NCHUNK_ = NTOK_ // CHUNK_
TILE_ = 128
NTILE_ = (NTOK_ * 2 + NEXP_ * (TILE_ - 1) + TILE_ - 1) // TILE_ + 1
NSLOT_ = NTILE_ * TILE_

_SC_INFO = plsc.get_sparse_core_info()
_NC = _SC_INFO.num_cores
_NS = _SC_INFO.num_subcores
_NW = _NC * _NS
_TPW = NTOK_ // _NW
_SCCH = 64


def _router_body(x_ref, rw1_ref, rb1_ref, rw2_ref, rb2_ref,
                 e1_ref, e2_ref, p1_ref, p2_ref):
    xc = x_ref[...]
    h = jnp.dot(xc, rw1_ref[...], preferred_element_type=jnp.float32)
    h = h + rb1_ref[0, :]
    h = h * jax.nn.sigmoid(h)
    logits = jnp.dot(h, rw2_ref[...], preferred_element_type=jnp.float32)
    logits = logits + rb2_ref[0, :]
    t = logits.shape[0]
    ii = lax.broadcasted_iota(jnp.int32, (t, NEXP_), 1)
    big = jnp.int32(127)
    m1 = jnp.max(logits, axis=1, keepdims=True)
    i1 = jnp.min(jnp.where(logits == m1, ii, big), axis=1, keepdims=True)
    neg = jnp.float32(-3.4e38)
    rest = jnp.where(ii == i1, neg, logits)
    m2 = jnp.max(rest, axis=1, keepdims=True)
    i2 = jnp.min(jnp.where(rest == m2, ii, big), axis=1, keepdims=True)
    e2v = jnp.exp(m2 - m1)
    p1 = 1.0 / (1.0 + e2v)
    e1_ref[...] = i1
    e2_ref[...] = i2
    p1_ref[...] = p1
    p2_ref[...] = e2v * p1


def _book_body(e1_ref, e2_ref, s1_ref, s2_ref, te_ref,
               cnt1_s, cnt2_s, rk1_s, rk2_s, off1_s, off2_s):
    ph = pl.program_id(0)
    c = pl.program_id(1)
    sl = pl.ds(c * CHUNK_, CHUNK_)
    ii8 = lax.broadcasted_iota(jnp.int32, (CHUNK_, NEXP_), 1)
    oh1 = (ii8 == e1_ref[...]).astype(jnp.float32)
    oh2 = (ii8 == e2_ref[...]).astype(jnp.float32)

    @pl.when(ph == 0)
    def _count_rank():
        @pl.when(c == 0)
        def _init():
            cnt1_s[...] = jnp.zeros((1, NEXP_), jnp.float32)
            cnt2_s[...] = jnp.zeros((1, NEXP_), jnp.float32)

        r = lax.broadcasted_iota(jnp.int32, (CHUNK_, CHUNK_), 0)
        q = lax.broadcasted_iota(jnp.int32, (CHUNK_, CHUNK_), 1)
        tri = (q < r).astype(jnp.float32)
        pre1 = jnp.dot(tri, oh1, preferred_element_type=jnp.float32)
        pre2 = jnp.dot(tri, oh2, preferred_element_type=jnp.float32)
        rk1 = jnp.sum(oh1 * (pre1 + cnt1_s[...]), axis=1, keepdims=True)
        rk2 = jnp.sum(oh2 * (pre2 + cnt2_s[...]), axis=1, keepdims=True)
        rk1_s[sl, :] = rk1
        rk2_s[sl, :] = rk2
        cnt1_s[...] = cnt1_s[...] + jnp.sum(oh1, axis=0, keepdims=True)
        cnt2_s[...] = cnt2_s[...] + jnp.sum(oh2, axis=0, keepdims=True)

    @pl.when(ph == 1)
    def _slots():
        @pl.when(c == 0)
        def _offsets():
            total = cnt1_s[...] + cnt2_s[...]
            padded = jnp.floor((total + (TILE_ - 1)) * (1.0 / TILE_)) * TILE_
            ea = lax.broadcasted_iota(jnp.int32, (NEXP_, NEXP_), 0)
            eb = lax.broadcasted_iota(jnp.int32, (NEXP_, NEXP_), 1)
            sut = (ea < eb).astype(jnp.float32)
            off = jnp.dot(padded, sut, preferred_element_type=jnp.float32)
            off1_s[...] = off
            off2_s[...] = off + cnt1_s[...]
            endv = off + padded
            ti = (lax.broadcasted_iota(jnp.int32, (NTILE_, NEXP_), 0)
                  * TILE_).astype(jnp.float32)
            tex = jnp.sum((ti >= endv).astype(jnp.float32), axis=1,
                          keepdims=True)
            te_ref[...] = jnp.minimum(tex, NEXP_ - 1).astype(jnp.int32)

        s1 = jnp.sum(oh1 * off1_s[...], axis=1, keepdims=True) + rk1_s[sl, :]
        s2 = jnp.sum(oh2 * off2_s[...], axis=1, keepdims=True) + rk2_s[sl, :]
        s1_ref[...] = s1.astype(jnp.int32)
        s2_ref[...] = s2.astype(jnp.int32)


def _dispatch_sc(x_hbm, s1_hbm, s2_hbm, xs_hbm, rows_v, i1_v, i2_v,
                 sem1, sem2):
    wid = lax.axis_index("s") * _NC + lax.axis_index("c")
    for ch in range(_TPW // _SCCH):
        base = wid * _TPW + ch * _SCCH
        pltpu.sync_copy(x_hbm.at[pl.ds(base, _SCCH)], rows_v)
        pltpu.sync_copy(s1_hbm.at[pl.ds(base, _SCCH)], i1_v)
        pltpu.sync_copy(s2_hbm.at[pl.ds(base, _SCCH)], i2_v)
        c1 = pltpu.async_copy(rows_v, xs_hbm.at[i1_v], sem1)
        c2 = pltpu.async_copy(rows_v, xs_hbm.at[i2_v], sem2)
        c1.wait()
        c2.wait()


def _ffn_body(te_ref, xs_ref, ew1_ref, eb1_ref, ew2_ref, eb2_ref, y_ref):
    xb = xs_ref[...].astype(jnp.bfloat16)
    h = jnp.dot(xb, ew1_ref[0], preferred_element_type=jnp.float32)
    h = h + eb1_ref[0, 0, :]
    h = h * jax.nn.sigmoid(h)
    y = jnp.dot(h.astype(jnp.bfloat16), ew2_ref[0],
                preferred_element_type=jnp.float32)
    y_ref[...] = y + eb2_ref[0, 0, :]


def _collect_sc(y_hbm, s1_hbm, s2_hbm, g1_hbm, g2_hbm, buf_v, i1_v, i2_v,
                sem):
    wid = lax.axis_index("s") * _NC + lax.axis_index("c")
    for ch in range(_TPW // _SCCH):
        base = wid * _TPW + ch * _SCCH
        pltpu.sync_copy(s1_hbm.at[pl.ds(base, _SCCH)], i1_v)
        pltpu.sync_copy(s2_hbm.at[pl.ds(base, _SCCH)], i2_v)
        pltpu.async_copy(y_hbm.at[i1_v], buf_v, sem).wait()
        pltpu.sync_copy(buf_v, g1_hbm.at[pl.ds(base, _SCCH)])
        pltpu.async_copy(y_hbm.at[i2_v], buf_v, sem).wait()
        pltpu.sync_copy(buf_v, g2_hbm.at[pl.ds(base, _SCCH)])


def _combine_body(g1_ref, g2_ref, p1_ref, p2_ref, out_ref):
    out_ref[...] = g1_ref[...] * p1_ref[...] + g2_ref[...] * p2_ref[...]


def kernel(input, rw1, rb1, rw2, rb2, ew1, eb1, ew2, eb2):
    bq, sq, d = input.shape
    x = input.reshape(bq * sq, d)

    e1, e2, p1, p2 = pl.pallas_call(
        _router_body,
        grid=(NCHUNK_,),
        in_specs=[
            pl.BlockSpec((CHUNK_, EMB_), lambda c: (c, 0)),
            pl.BlockSpec((EMB_, HID_), lambda c: (0, 0)),
            pl.BlockSpec((1, HID_), lambda c: (0, 0)),
            pl.BlockSpec((HID_, NEXP_), lambda c: (0, 0)),
            pl.BlockSpec((1, NEXP_), lambda c: (0, 0)),
        ],
        out_specs=[
            pl.BlockSpec((CHUNK_, 1), lambda c: (c, 0)),
            pl.BlockSpec((CHUNK_, 1), lambda c: (c, 0)),
            pl.BlockSpec((CHUNK_, 1), lambda c: (c, 0)),
            pl.BlockSpec((CHUNK_, 1), lambda c: (c, 0)),
        ],
        out_shape=[
            jax.ShapeDtypeStruct((NTOK_, 1), jnp.int32),
            jax.ShapeDtypeStruct((NTOK_, 1), jnp.int32),
            jax.ShapeDtypeStruct((NTOK_, 1), jnp.float32),
            jax.ShapeDtypeStruct((NTOK_, 1), jnp.float32),
        ],
    )(x, rw1, rb1.reshape(1, HID_), rw2, rb2.reshape(1, NEXP_))

    s1, s2, te = pl.pallas_call(
        _book_body,
        grid=(2, NCHUNK_),
        in_specs=[
            pl.BlockSpec((CHUNK_, 1), lambda p, c: (c, 0)),
            pl.BlockSpec((CHUNK_, 1), lambda p, c: (c, 0)),
        ],
        out_specs=[
            pl.BlockSpec((CHUNK_, 1), lambda p, c: (c, 0)),
            pl.BlockSpec((CHUNK_, 1), lambda p, c: (c, 0)),
            pl.BlockSpec((NTILE_, 1), lambda p, c: (0, 0)),
        ],
        out_shape=[
            jax.ShapeDtypeStruct((NTOK_, 1), jnp.int32),
            jax.ShapeDtypeStruct((NTOK_, 1), jnp.int32),
            jax.ShapeDtypeStruct((NTILE_, 1), jnp.int32),
        ],
        scratch_shapes=[
            pltpu.VMEM((1, NEXP_), jnp.float32),
            pltpu.VMEM((1, NEXP_), jnp.float32),
            pltpu.VMEM((NTOK_, 1), jnp.float32),
            pltpu.VMEM((NTOK_, 1), jnp.float32),
            pltpu.VMEM((1, NEXP_), jnp.float32),
            pltpu.VMEM((1, NEXP_), jnp.float32),
        ],
    )(e1, e2)

    s1f = s1.reshape(NTOK_)
    s2f = s2.reshape(NTOK_)
    tef = te.reshape(NTILE_)

    mesh = plsc.VectorSubcoreMesh(core_axis_name="c", subcore_axis_name="s")

    dispatch = pl.kernel(
        _dispatch_sc,
        mesh=mesh,
        out_type=jax.ShapeDtypeStruct((NSLOT_, EMB_), jnp.float32),
        scratch_types=[
            pltpu.VMEM((_SCCH, EMB_), jnp.float32),
            pltpu.VMEM((_SCCH,), jnp.int32),
            pltpu.VMEM((_SCCH,), jnp.int32),
            pltpu.SemaphoreType.DMA,
            pltpu.SemaphoreType.DMA,
        ],
    )
    xs = dispatch(x, s1f, s2f)

    y = pl.pallas_call(
        _ffn_body,
        grid_spec=pltpu.PrefetchScalarGridSpec(
            num_scalar_prefetch=1,
            grid=(NTILE_,),
            in_specs=[
                pl.BlockSpec((TILE_, EMB_), lambda i, te_r: (i, 0)),
                pl.BlockSpec((1, EMB_, HID_), lambda i, te_r: (te_r[i], 0, 0)),
                pl.BlockSpec((1, 1, HID_), lambda i, te_r: (te_r[i], 0, 0)),
                pl.BlockSpec((1, HID_, EMB_), lambda i, te_r: (te_r[i], 0, 0)),
                pl.BlockSpec((1, 1, EMB_), lambda i, te_r: (te_r[i], 0, 0)),
            ],
            out_specs=pl.BlockSpec((TILE_, EMB_), lambda i, te_r: (i, 0)),
        ),
        out_shape=jax.ShapeDtypeStruct((NSLOT_, EMB_), jnp.float32),
    )(tef, xs, ew1.astype(jnp.bfloat16), eb1.reshape(NEXP_, 1, HID_),
      ew2.astype(jnp.bfloat16), eb2.reshape(NEXP_, 1, EMB_))

    collect = pl.kernel(
        _collect_sc,
        mesh=mesh,
        out_type=[
            jax.ShapeDtypeStruct((NTOK_, EMB_), jnp.float32),
            jax.ShapeDtypeStruct((NTOK_, EMB_), jnp.float32),
        ],
        scratch_types=[
            pltpu.VMEM((_SCCH, EMB_), jnp.float32),
            pltpu.VMEM((_SCCH,), jnp.int32),
            pltpu.VMEM((_SCCH,), jnp.int32),
            pltpu.SemaphoreType.DMA,
        ],
    )
    g1, g2 = collect(y, s1f, s2f)

    out = pl.pallas_call(
        _combine_body,
        grid=(NCHUNK_,),
        in_specs=[
            pl.BlockSpec((CHUNK_, EMB_), lambda c: (c, 0)),
            pl.BlockSpec((CHUNK_, EMB_), lambda c: (c, 0)),
            pl.BlockSpec((CHUNK_, 1), lambda c: (c, 0)),
            pl.BlockSpec((CHUNK_, 1), lambda c: (c, 0)),
        ],
        out_specs=pl.BlockSpec((CHUNK_, EMB_), lambda c: (c, 0)),
        out_shape=jax.ShapeDtypeStruct((NTOK_, EMB_), jnp.float32),
    )(g1, g2, p1, p2)

    return out.reshape(bq, sq, d)


# speedup vs baseline: 1.0295x; 1.0295x over previous
import functools

import jax
import jax.numpy as jnp
from jax import lax
from jax.experimental import pallas as pl
from jax.experimental.pallas import tpu as pltpu
from jax.experimental.pallas import tpu_sc as plsc

EMB_ = 1024
HID_ = 2048
NEXP_ = 8
NTOK_ = 4096
CHUNK_ = 512
NCHUNK_ = NTOK_ // CHUNK_
TILE_ = 256
NTILE_ = (NTOK_ * 2 + NEXP_ * (TILE_ - 1) + TILE_ - 1) // TILE_ + 1
NSLOT_ = NTILE_ * TILE_

_SC_INFO = plsc.get_sparse_core_info()
_NC = _SC_INFO.num_cores
_NS = _SC_INFO.num_subcores
_NW = _NC * _NS
_TPW = NTOK_ // _NW
_SCCH = 64


def _router_body(x_ref, rw1_ref, rb1_ref, rw2_ref, rb2_ref,
                 e1_ref, e2_ref, p1_ref, p2_ref):
    xc = x_ref[...]
    h = jnp.dot(xc, rw1_ref[...], preferred_element_type=jnp.float32)
    h = h + rb1_ref[0, :]
    h = h * jax.nn.sigmoid(h)
    logits = jnp.dot(h, rw2_ref[...], preferred_element_type=jnp.float32)
    logits = logits + rb2_ref[0, :]
    t = logits.shape[0]
    ii = lax.broadcasted_iota(jnp.int32, (t, NEXP_), 1)
    big = jnp.int32(127)
    m1 = jnp.max(logits, axis=1, keepdims=True)
    i1 = jnp.min(jnp.where(logits == m1, ii, big), axis=1, keepdims=True)
    neg = jnp.float32(-3.4e38)
    rest = jnp.where(ii == i1, neg, logits)
    m2 = jnp.max(rest, axis=1, keepdims=True)
    i2 = jnp.min(jnp.where(rest == m2, ii, big), axis=1, keepdims=True)
    e2v = jnp.exp(m2 - m1)
    p1 = 1.0 / (1.0 + e2v)
    e1_ref[...] = i1
    e2_ref[...] = i2
    p1_ref[...] = p1
    p2_ref[...] = e2v * p1


def _book_body(e1_ref, e2_ref, s1_ref, s2_ref, te_ref,
               cnt1_s, cnt2_s, rk1_s, rk2_s, off1_s, off2_s):
    ph = pl.program_id(0)
    c = pl.program_id(1)
    sl = pl.ds(c * CHUNK_, CHUNK_)
    ii8 = lax.broadcasted_iota(jnp.int32, (CHUNK_, NEXP_), 1)
    oh1 = (ii8 == e1_ref[...]).astype(jnp.float32)
    oh2 = (ii8 == e2_ref[...]).astype(jnp.float32)

    @pl.when(ph == 0)
    def _count_rank():
        @pl.when(c == 0)
        def _init():
            cnt1_s[...] = jnp.zeros((1, NEXP_), jnp.float32)
            cnt2_s[...] = jnp.zeros((1, NEXP_), jnp.float32)

        r = lax.broadcasted_iota(jnp.int32, (CHUNK_, CHUNK_), 0)
        q = lax.broadcasted_iota(jnp.int32, (CHUNK_, CHUNK_), 1)
        tri = (q < r).astype(jnp.float32)
        pre1 = jnp.dot(tri, oh1, preferred_element_type=jnp.float32)
        pre2 = jnp.dot(tri, oh2, preferred_element_type=jnp.float32)
        rk1 = jnp.sum(oh1 * (pre1 + cnt1_s[...]), axis=1, keepdims=True)
        rk2 = jnp.sum(oh2 * (pre2 + cnt2_s[...]), axis=1, keepdims=True)
        rk1_s[sl, :] = rk1
        rk2_s[sl, :] = rk2
        cnt1_s[...] = cnt1_s[...] + jnp.sum(oh1, axis=0, keepdims=True)
        cnt2_s[...] = cnt2_s[...] + jnp.sum(oh2, axis=0, keepdims=True)

    @pl.when(ph == 1)
    def _slots():
        @pl.when(c == 0)
        def _offsets():
            total = cnt1_s[...] + cnt2_s[...]
            padded = jnp.floor((total + (TILE_ - 1)) * (1.0 / TILE_)) * TILE_
            ea = lax.broadcasted_iota(jnp.int32, (NEXP_, NEXP_), 0)
            eb = lax.broadcasted_iota(jnp.int32, (NEXP_, NEXP_), 1)
            sut = (ea < eb).astype(jnp.float32)
            off = jnp.dot(padded, sut, preferred_element_type=jnp.float32)
            off1_s[...] = off
            off2_s[...] = off + cnt1_s[...]
            endv = off + padded
            ti = (lax.broadcasted_iota(jnp.int32, (NTILE_, NEXP_), 0)
                  * TILE_).astype(jnp.float32)
            tex = jnp.sum((ti >= endv).astype(jnp.float32), axis=1,
                          keepdims=True)
            te_ref[...] = jnp.minimum(tex, NEXP_ - 1).astype(jnp.int32)

        s1 = jnp.sum(oh1 * off1_s[...], axis=1, keepdims=True) + rk1_s[sl, :]
        s2 = jnp.sum(oh2 * off2_s[...], axis=1, keepdims=True) + rk2_s[sl, :]
        s1_ref[...] = s1.astype(jnp.int32)
        s2_ref[...] = s2.astype(jnp.int32)


def _dispatch_sc(x_hbm, s1_hbm, s2_hbm, xs_hbm, rows_v, i1_v, i2_v,
                 sem1, sem2):
    wid = lax.axis_index("s") * _NC + lax.axis_index("c")
    for ch in range(_TPW // _SCCH):
        base = wid * _TPW + ch * _SCCH
        pltpu.sync_copy(x_hbm.at[pl.ds(base, _SCCH)], rows_v)
        pltpu.sync_copy(s1_hbm.at[pl.ds(base, _SCCH)], i1_v)
        pltpu.sync_copy(s2_hbm.at[pl.ds(base, _SCCH)], i2_v)
        c1 = pltpu.async_copy(rows_v, xs_hbm.at[i1_v], sem1)
        c2 = pltpu.async_copy(rows_v, xs_hbm.at[i2_v], sem2)
        c1.wait()
        c2.wait()


def _ffn_body(te_ref, xs_ref, ew1_ref, eb1_ref, ew2_ref, eb2_ref, y_ref):
    xb = xs_ref[...].astype(jnp.bfloat16)
    h = jnp.dot(xb, ew1_ref[0], preferred_element_type=jnp.float32)
    h = h + eb1_ref[0, 0, :]
    h = h * jax.nn.sigmoid(h)
    y = jnp.dot(h.astype(jnp.bfloat16), ew2_ref[0],
                preferred_element_type=jnp.float32)
    y_ref[...] = y + eb2_ref[0, 0, :]


def _collect_sc(y_hbm, s1_hbm, s2_hbm, g1_hbm, g2_hbm, buf_v, i1_v, i2_v,
                sem):
    wid = lax.axis_index("s") * _NC + lax.axis_index("c")
    for ch in range(_TPW // _SCCH):
        base = wid * _TPW + ch * _SCCH
        pltpu.sync_copy(s1_hbm.at[pl.ds(base, _SCCH)], i1_v)
        pltpu.sync_copy(s2_hbm.at[pl.ds(base, _SCCH)], i2_v)
        pltpu.async_copy(y_hbm.at[i1_v], buf_v, sem).wait()
        pltpu.sync_copy(buf_v, g1_hbm.at[pl.ds(base, _SCCH)])
        pltpu.async_copy(y_hbm.at[i2_v], buf_v, sem).wait()
        pltpu.sync_copy(buf_v, g2_hbm.at[pl.ds(base, _SCCH)])


def _combine_body(g1_ref, g2_ref, p1_ref, p2_ref, out_ref):
    out_ref[...] = g1_ref[...] * p1_ref[...] + g2_ref[...] * p2_ref[...]


def kernel(input, rw1, rb1, rw2, rb2, ew1, eb1, ew2, eb2):
    bq, sq, d = input.shape
    x = input.reshape(bq * sq, d)

    e1, e2, p1, p2 = pl.pallas_call(
        _router_body,
        grid=(NCHUNK_,),
        in_specs=[
            pl.BlockSpec((CHUNK_, EMB_), lambda c: (c, 0)),
            pl.BlockSpec((EMB_, HID_), lambda c: (0, 0)),
            pl.BlockSpec((1, HID_), lambda c: (0, 0)),
            pl.BlockSpec((HID_, NEXP_), lambda c: (0, 0)),
            pl.BlockSpec((1, NEXP_), lambda c: (0, 0)),
        ],
        out_specs=[
            pl.BlockSpec((CHUNK_, 1), lambda c: (c, 0)),
            pl.BlockSpec((CHUNK_, 1), lambda c: (c, 0)),
            pl.BlockSpec((CHUNK_, 1), lambda c: (c, 0)),
            pl.BlockSpec((CHUNK_, 1), lambda c: (c, 0)),
        ],
        out_shape=[
            jax.ShapeDtypeStruct((NTOK_, 1), jnp.int32),
            jax.ShapeDtypeStruct((NTOK_, 1), jnp.int32),
            jax.ShapeDtypeStruct((NTOK_, 1), jnp.float32),
            jax.ShapeDtypeStruct((NTOK_, 1), jnp.float32),
        ],
    )(x, rw1, rb1.reshape(1, HID_), rw2, rb2.reshape(1, NEXP_))

    s1, s2, te = pl.pallas_call(
        _book_body,
        grid=(2, NCHUNK_),
        in_specs=[
            pl.BlockSpec((CHUNK_, 1), lambda p, c: (c, 0)),
            pl.BlockSpec((CHUNK_, 1), lambda p, c: (c, 0)),
        ],
        out_specs=[
            pl.BlockSpec((CHUNK_, 1), lambda p, c: (c, 0)),
            pl.BlockSpec((CHUNK_, 1), lambda p, c: (c, 0)),
            pl.BlockSpec((NTILE_, 1), lambda p, c: (0, 0)),
        ],
        out_shape=[
            jax.ShapeDtypeStruct((NTOK_, 1), jnp.int32),
            jax.ShapeDtypeStruct((NTOK_, 1), jnp.int32),
            jax.ShapeDtypeStruct((NTILE_, 1), jnp.int32),
        ],
        scratch_shapes=[
            pltpu.VMEM((1, NEXP_), jnp.float32),
            pltpu.VMEM((1, NEXP_), jnp.float32),
            pltpu.VMEM((NTOK_, 1), jnp.float32),
            pltpu.VMEM((NTOK_, 1), jnp.float32),
            pltpu.VMEM((1, NEXP_), jnp.float32),
            pltpu.VMEM((1, NEXP_), jnp.float32),
        ],
    )(e1, e2)

    s1f = s1.reshape(NTOK_)
    s2f = s2.reshape(NTOK_)
    tef = te.reshape(NTILE_)

    mesh = plsc.VectorSubcoreMesh(core_axis_name="c", subcore_axis_name="s")

    dispatch = pl.kernel(
        _dispatch_sc,
        mesh=mesh,
        out_type=jax.ShapeDtypeStruct((NSLOT_, EMB_), jnp.float32),
        scratch_types=[
            pltpu.VMEM((_SCCH, EMB_), jnp.float32),
            pltpu.VMEM((_SCCH,), jnp.int32),
            pltpu.VMEM((_SCCH,), jnp.int32),
            pltpu.SemaphoreType.DMA,
            pltpu.SemaphoreType.DMA,
        ],
    )
    xs = dispatch(x, s1f, s2f)

    y = pl.pallas_call(
        _ffn_body,
        grid_spec=pltpu.PrefetchScalarGridSpec(
            num_scalar_prefetch=1,
            grid=(NTILE_,),
            in_specs=[
                pl.BlockSpec((TILE_, EMB_), lambda i, te_r: (i, 0)),
                pl.BlockSpec((1, EMB_, HID_), lambda i, te_r: (te_r[i], 0, 0)),
                pl.BlockSpec((1, 1, HID_), lambda i, te_r: (te_r[i], 0, 0)),
                pl.BlockSpec((1, HID_, EMB_), lambda i, te_r: (te_r[i], 0, 0)),
                pl.BlockSpec((1, 1, EMB_), lambda i, te_r: (te_r[i], 0, 0)),
            ],
            out_specs=pl.BlockSpec((TILE_, EMB_), lambda i, te_r: (i, 0)),
        ),
        out_shape=jax.ShapeDtypeStruct((NSLOT_, EMB_), jnp.float32),
    )(tef, xs, ew1.astype(jnp.bfloat16), eb1.reshape(NEXP_, 1, HID_),
      ew2.astype(jnp.bfloat16), eb2.reshape(NEXP_, 1, EMB_))

    collect = pl.kernel(
        _collect_sc,
        mesh=mesh,
        out_type=[
            jax.ShapeDtypeStruct((NTOK_, EMB_), jnp.float32),
            jax.ShapeDtypeStruct((NTOK_, EMB_), jnp.float32),
        ],
        scratch_types=[
            pltpu.VMEM((_SCCH, EMB_), jnp.float32),
            pltpu.VMEM((_SCCH,), jnp.int32),
            pltpu.VMEM((_SCCH,), jnp.int32),
            pltpu.SemaphoreType.DMA,
        ],
    )
    g1, g2 = collect(y, s1f, s2f)

    out = pl.pallas_call(
        _combine_body,
        grid=(NCHUNK_,),
        in_specs=[
            pl.BlockSpec((CHUNK_, EMB_), lambda c: (c, 0)),
            pl.BlockSpec((CHUNK_, EMB_), lambda c: (c, 0)),
            pl.BlockSpec((CHUNK_, 1), lambda c: (c, 0)),
            pl.BlockSpec((CHUNK_, 1), lambda c: (c, 0)),
        ],
        out_specs=pl.BlockSpec((CHUNK_, EMB_), lambda c: (c, 0)),
        out_shape=jax.ShapeDtypeStruct((NTOK_, EMB_), jnp.float32),
    )(g1, g2, p1, p2)

    return out.reshape(bq, sq, d)


# speedup vs baseline: 1.1900x; 1.1559x over previous
import functools

import jax
import jax.numpy as jnp
from jax import lax
from jax.experimental import pallas as pl
from jax.experimental.pallas import tpu as pltpu
from jax.experimental.pallas import tpu_sc as plsc

EMB_ = 1024
HID_ = 2048
NEXP_ = 8
NTOK_ = 4096
CHUNK_ = 512
NCHUNK_ = NTOK_ // CHUNK_
TILE_ = 256
NTILE_ = (NTOK_ * 2 + NEXP_ * (TILE_ - 1) + TILE_ - 1) // TILE_ + 1
NSLOT_ = NTILE_ * TILE_

_SC_INFO = plsc.get_sparse_core_info()
_NC = _SC_INFO.num_cores
_NS = _SC_INFO.num_subcores
_NW = _NC * _NS
_TPW = NTOK_ // _NW
_SCCH = 64
_GCH = 32


def _router_body(x_ref, rw1_ref, rb1_ref, rw2_ref, rb2_ref,
                 e1_ref, e2_ref, p1_ref, p2_ref):
    xc = x_ref[...]
    h = jnp.dot(xc, rw1_ref[...], preferred_element_type=jnp.float32)
    h = h + rb1_ref[0, :]
    h = h * jax.nn.sigmoid(h)
    logits = jnp.dot(h, rw2_ref[...], preferred_element_type=jnp.float32)
    logits = logits + rb2_ref[0, :]
    t = logits.shape[0]
    ii = lax.broadcasted_iota(jnp.int32, (t, NEXP_), 1)
    big = jnp.int32(127)
    m1 = jnp.max(logits, axis=1, keepdims=True)
    i1 = jnp.min(jnp.where(logits == m1, ii, big), axis=1, keepdims=True)
    neg = jnp.float32(-3.4e38)
    rest = jnp.where(ii == i1, neg, logits)
    m2 = jnp.max(rest, axis=1, keepdims=True)
    i2 = jnp.min(jnp.where(rest == m2, ii, big), axis=1, keepdims=True)
    e2v = jnp.exp(m2 - m1)
    p1 = 1.0 / (1.0 + e2v)
    e1_ref[...] = i1
    e2_ref[...] = i2
    p1_ref[...] = p1
    p2_ref[...] = e2v * p1


def _book_body(e1_ref, e2_ref, s1_ref, s2_ref, te_ref,
               cnt1_s, cnt2_s, rk1_s, rk2_s, off1_s, off2_s):
    ph = pl.program_id(0)
    c = pl.program_id(1)
    sl = pl.ds(c * CHUNK_, CHUNK_)
    ii8 = lax.broadcasted_iota(jnp.int32, (CHUNK_, NEXP_), 1)
    oh1 = (ii8 == e1_ref[...]).astype(jnp.float32)
    oh2 = (ii8 == e2_ref[...]).astype(jnp.float32)

    @pl.when(ph == 0)
    def _count_rank():
        @pl.when(c == 0)
        def _init():
            cnt1_s[...] = jnp.zeros((1, NEXP_), jnp.float32)
            cnt2_s[...] = jnp.zeros((1, NEXP_), jnp.float32)

        r = lax.broadcasted_iota(jnp.int32, (CHUNK_, CHUNK_), 0)
        q = lax.broadcasted_iota(jnp.int32, (CHUNK_, CHUNK_), 1)
        tri = (q < r).astype(jnp.float32)
        pre1 = jnp.dot(tri, oh1, preferred_element_type=jnp.float32)
        pre2 = jnp.dot(tri, oh2, preferred_element_type=jnp.float32)
        rk1 = jnp.sum(oh1 * (pre1 + cnt1_s[...]), axis=1, keepdims=True)
        rk2 = jnp.sum(oh2 * (pre2 + cnt2_s[...]), axis=1, keepdims=True)
        rk1_s[sl, :] = rk1
        rk2_s[sl, :] = rk2
        cnt1_s[...] = cnt1_s[...] + jnp.sum(oh1, axis=0, keepdims=True)
        cnt2_s[...] = cnt2_s[...] + jnp.sum(oh2, axis=0, keepdims=True)

    @pl.when(ph == 1)
    def _slots():
        @pl.when(c == 0)
        def _offsets():
            total = cnt1_s[...] + cnt2_s[...]
            padded = jnp.floor((total + (TILE_ - 1)) * (1.0 / TILE_)) * TILE_
            ea = lax.broadcasted_iota(jnp.int32, (NEXP_, NEXP_), 0)
            eb = lax.broadcasted_iota(jnp.int32, (NEXP_, NEXP_), 1)
            sut = (ea < eb).astype(jnp.float32)
            off = jnp.dot(padded, sut, preferred_element_type=jnp.float32)
            off1_s[...] = off
            off2_s[...] = off + cnt1_s[...]
            endv = off + padded
            ti = (lax.broadcasted_iota(jnp.int32, (NTILE_, NEXP_), 0)
                  * TILE_).astype(jnp.float32)
            tex = jnp.sum((ti >= endv).astype(jnp.float32), axis=1,
                          keepdims=True)
            te_ref[...] = tex.astype(jnp.int32)

        s1 = jnp.sum(oh1 * off1_s[...], axis=1, keepdims=True) + rk1_s[sl, :]
        s2 = jnp.sum(oh2 * off2_s[...], axis=1, keepdims=True) + rk2_s[sl, :]
        s1_ref[...] = s1.astype(jnp.int32)
        s2_ref[...] = s2.astype(jnp.int32)


def _dispatch_sc(x_hbm, s1_hbm, s2_hbm, xs_hbm, rows_v, i1_v, i2_v,
                 sem1, sem2):
    wid = lax.axis_index("s") * _NC + lax.axis_index("c")
    for ch in range(_TPW // _SCCH):
        base = wid * _TPW + ch * _SCCH
        pltpu.sync_copy(x_hbm.at[pl.ds(base, _SCCH)], rows_v)
        pltpu.sync_copy(s1_hbm.at[pl.ds(base, _SCCH)], i1_v)
        pltpu.sync_copy(s2_hbm.at[pl.ds(base, _SCCH)], i2_v)
        c1 = pltpu.async_copy(rows_v, xs_hbm.at[i1_v], sem1)
        c2 = pltpu.async_copy(rows_v, xs_hbm.at[i2_v], sem2)
        c1.wait()
        c2.wait()


def _ffn_body(te_ref, xs_ref, ew1_ref, eb1_ref, ew2_ref, eb2_ref, y_ref):
    @pl.when(te_ref[pl.program_id(0)] < NEXP_)
    def _compute():
        h = jnp.dot(xs_ref[...], ew1_ref[0],
                    preferred_element_type=jnp.float32)
        h = h + eb1_ref[0, 0, :]
        h = h * jax.nn.sigmoid(h)
        y = jnp.dot(h, ew2_ref[0], preferred_element_type=jnp.float32)
        y_ref[...] = y + eb2_ref[0, 0, :]


def _collect_sc(y_hbm, s1_hbm, s2_hbm, g1_hbm, g2_hbm, b1_v, b2_v,
                i1_v, i2_v, sem1, sem2):
    wid = lax.axis_index("s") * _NC + lax.axis_index("c")
    for ch in range(_TPW // _GCH):
        base = wid * _TPW + ch * _GCH
        pltpu.sync_copy(s1_hbm.at[pl.ds(base, _GCH)], i1_v)
        pltpu.sync_copy(s2_hbm.at[pl.ds(base, _GCH)], i2_v)
        c1 = pltpu.async_copy(y_hbm.at[i1_v], b1_v, sem1)
        c2 = pltpu.async_copy(y_hbm.at[i2_v], b2_v, sem2)
        c1.wait()
        c2.wait()
        pltpu.sync_copy(b1_v, g1_hbm.at[pl.ds(base, _GCH)])
        pltpu.sync_copy(b2_v, g2_hbm.at[pl.ds(base, _GCH)])


def _combine_body(g1_ref, g2_ref, p1_ref, p2_ref, out_ref):
    out_ref[...] = g1_ref[...] * p1_ref[...] + g2_ref[...] * p2_ref[...]


def kernel(input, rw1, rb1, rw2, rb2, ew1, eb1, ew2, eb2):
    bq, sq, d = input.shape
    x = input.reshape(bq * sq, d)

    e1, e2, p1, p2 = pl.pallas_call(
        _router_body,
        grid=(NCHUNK_,),
        in_specs=[
            pl.BlockSpec((CHUNK_, EMB_), lambda c: (c, 0)),
            pl.BlockSpec((EMB_, HID_), lambda c: (0, 0)),
            pl.BlockSpec((1, HID_), lambda c: (0, 0)),
            pl.BlockSpec((HID_, NEXP_), lambda c: (0, 0)),
            pl.BlockSpec((1, NEXP_), lambda c: (0, 0)),
        ],
        out_specs=[
            pl.BlockSpec((CHUNK_, 1), lambda c: (c, 0)),
            pl.BlockSpec((CHUNK_, 1), lambda c: (c, 0)),
            pl.BlockSpec((CHUNK_, 1), lambda c: (c, 0)),
            pl.BlockSpec((CHUNK_, 1), lambda c: (c, 0)),
        ],
        out_shape=[
            jax.ShapeDtypeStruct((NTOK_, 1), jnp.int32),
            jax.ShapeDtypeStruct((NTOK_, 1), jnp.int32),
            jax.ShapeDtypeStruct((NTOK_, 1), jnp.float32),
            jax.ShapeDtypeStruct((NTOK_, 1), jnp.float32),
        ],
    )(x, rw1, rb1.reshape(1, HID_), rw2, rb2.reshape(1, NEXP_))

    s1, s2, te = pl.pallas_call(
        _book_body,
        grid=(2, NCHUNK_),
        in_specs=[
            pl.BlockSpec((CHUNK_, 1), lambda p, c: (c, 0)),
            pl.BlockSpec((CHUNK_, 1), lambda p, c: (c, 0)),
        ],
        out_specs=[
            pl.BlockSpec((CHUNK_, 1), lambda p, c: (c, 0)),
            pl.BlockSpec((CHUNK_, 1), lambda p, c: (c, 0)),
            pl.BlockSpec((NTILE_, 1), lambda p, c: (0, 0)),
        ],
        out_shape=[
            jax.ShapeDtypeStruct((NTOK_, 1), jnp.int32),
            jax.ShapeDtypeStruct((NTOK_, 1), jnp.int32),
            jax.ShapeDtypeStruct((NTILE_, 1), jnp.int32),
        ],
        scratch_shapes=[
            pltpu.VMEM((1, NEXP_), jnp.float32),
            pltpu.VMEM((1, NEXP_), jnp.float32),
            pltpu.VMEM((NTOK_, 1), jnp.float32),
            pltpu.VMEM((NTOK_, 1), jnp.float32),
            pltpu.VMEM((1, NEXP_), jnp.float32),
            pltpu.VMEM((1, NEXP_), jnp.float32),
        ],
    )(e1, e2)

    s1f = s1.reshape(NTOK_)
    s2f = s2.reshape(NTOK_)
    tef = te.reshape(NTILE_)

    mesh = plsc.VectorSubcoreMesh(core_axis_name="c", subcore_axis_name="s")

    dispatch = pl.kernel(
        _dispatch_sc,
        mesh=mesh,
        out_type=jax.ShapeDtypeStruct((NSLOT_, EMB_), jnp.float32),
        scratch_types=[
            pltpu.VMEM((_SCCH, EMB_), jnp.float32),
            pltpu.VMEM((_SCCH,), jnp.int32),
            pltpu.VMEM((_SCCH,), jnp.int32),
            pltpu.SemaphoreType.DMA,
            pltpu.SemaphoreType.DMA,
        ],
    )
    xs = dispatch(x, s1f, s2f)

    y = pl.pallas_call(
        _ffn_body,
        grid_spec=pltpu.PrefetchScalarGridSpec(
            num_scalar_prefetch=1,
            grid=(NTILE_,),
            in_specs=[
                pl.BlockSpec((TILE_, EMB_), lambda i, te_r: (i, 0)),
                pl.BlockSpec((1, EMB_, HID_),
                             lambda i, te_r: (jnp.minimum(te_r[i], NEXP_ - 1),
                                              0, 0)),
                pl.BlockSpec((1, 1, HID_),
                             lambda i, te_r: (jnp.minimum(te_r[i], NEXP_ - 1),
                                              0, 0)),
                pl.BlockSpec((1, HID_, EMB_),
                             lambda i, te_r: (jnp.minimum(te_r[i], NEXP_ - 1),
                                              0, 0)),
                pl.BlockSpec((1, 1, EMB_),
                             lambda i, te_r: (jnp.minimum(te_r[i], NEXP_ - 1),
                                              0, 0)),
            ],
            out_specs=pl.BlockSpec((TILE_, EMB_), lambda i, te_r: (i, 0)),
        ),
        out_shape=jax.ShapeDtypeStruct((NSLOT_, EMB_), jnp.float32),
    )(tef, xs, ew1, eb1.reshape(NEXP_, 1, HID_), ew2,
      eb2.reshape(NEXP_, 1, EMB_))

    collect = pl.kernel(
        _collect_sc,
        mesh=mesh,
        out_type=[
            jax.ShapeDtypeStruct((NTOK_, EMB_), jnp.float32),
            jax.ShapeDtypeStruct((NTOK_, EMB_), jnp.float32),
        ],
        scratch_types=[
            pltpu.VMEM((_GCH, EMB_), jnp.float32),
            pltpu.VMEM((_GCH, EMB_), jnp.float32),
            pltpu.VMEM((_GCH,), jnp.int32),
            pltpu.VMEM((_GCH,), jnp.int32),
            pltpu.SemaphoreType.DMA,
            pltpu.SemaphoreType.DMA,
        ],
    )
    g1, g2 = collect(y, s1f, s2f)

    out = pl.pallas_call(
        _combine_body,
        grid=(NCHUNK_,),
        in_specs=[
            pl.BlockSpec((CHUNK_, EMB_), lambda c: (c, 0)),
            pl.BlockSpec((CHUNK_, EMB_), lambda c: (c, 0)),
            pl.BlockSpec((CHUNK_, 1), lambda c: (c, 0)),
            pl.BlockSpec((CHUNK_, 1), lambda c: (c, 0)),
        ],
        out_specs=pl.BlockSpec((CHUNK_, EMB_), lambda c: (c, 0)),
        out_shape=jax.ShapeDtypeStruct((NTOK_, EMB_), jnp.float32),
    )(g1, g2, p1, p2)

    return out.reshape(bq, sq, d)


# speedup vs baseline: 1.2027x; 1.0107x over previous
import functools

import jax
import jax.numpy as jnp
from jax import lax
from jax.experimental import pallas as pl
from jax.experimental.pallas import tpu as pltpu
from jax.experimental.pallas import tpu_sc as plsc

EMB_ = 1024
HID_ = 2048
NEXP_ = 8
NTOK_ = 4096
CHUNK_ = 512
NCHUNK_ = NTOK_ // CHUNK_
TILE_ = 256
NTILE_ = (NTOK_ * 2 + NEXP_ * (TILE_ - 1) + TILE_ - 1) // TILE_ + 1
NSLOT_ = NTILE_ * TILE_

_SC_INFO = plsc.get_sparse_core_info()
_NC = _SC_INFO.num_cores
_NS = _SC_INFO.num_subcores
_NW = _NC * _NS
_TPW = NTOK_ // _NW
_SCCH = 64


def _router_body(x_ref, rw1_ref, rb1_ref, rw2_ref, rb2_ref,
                 e1_ref, e2_ref, p1_ref, p2_ref):
    xc = x_ref[...]
    h = jnp.dot(xc, rw1_ref[...], preferred_element_type=jnp.float32)
    h = h + rb1_ref[0, :]
    h = h * jax.nn.sigmoid(h)
    logits = jnp.dot(h, rw2_ref[...], preferred_element_type=jnp.float32)
    logits = logits + rb2_ref[0, :]
    t = logits.shape[0]
    ii = lax.broadcasted_iota(jnp.int32, (t, NEXP_), 1)
    big = jnp.int32(127)
    m1 = jnp.max(logits, axis=1, keepdims=True)
    i1 = jnp.min(jnp.where(logits == m1, ii, big), axis=1, keepdims=True)
    neg = jnp.float32(-3.4e38)
    rest = jnp.where(ii == i1, neg, logits)
    m2 = jnp.max(rest, axis=1, keepdims=True)
    i2 = jnp.min(jnp.where(rest == m2, ii, big), axis=1, keepdims=True)
    e2v = jnp.exp(m2 - m1)
    p1 = 1.0 / (1.0 + e2v)
    e1_ref[...] = i1
    e2_ref[...] = i2
    p1_ref[...] = p1
    p2_ref[...] = e2v * p1


def _book_body(e1_ref, e2_ref, s1_ref, s2_ref, te_ref,
               cnt1_s, cnt2_s, rk1_s, rk2_s, off1_s, off2_s):
    ph = pl.program_id(0)
    c = pl.program_id(1)
    sl = pl.ds(c * CHUNK_, CHUNK_)
    ii8 = lax.broadcasted_iota(jnp.int32, (CHUNK_, NEXP_), 1)
    oh1 = (ii8 == e1_ref[...]).astype(jnp.float32)
    oh2 = (ii8 == e2_ref[...]).astype(jnp.float32)

    @pl.when(ph == 0)
    def _count_rank():
        @pl.when(c == 0)
        def _init():
            cnt1_s[...] = jnp.zeros((1, NEXP_), jnp.float32)
            cnt2_s[...] = jnp.zeros((1, NEXP_), jnp.float32)

        r = lax.broadcasted_iota(jnp.int32, (CHUNK_, CHUNK_), 0)
        q = lax.broadcasted_iota(jnp.int32, (CHUNK_, CHUNK_), 1)
        tri = (q < r).astype(jnp.float32)
        pre1 = jnp.dot(tri, oh1, preferred_element_type=jnp.float32)
        pre2 = jnp.dot(tri, oh2, preferred_element_type=jnp.float32)
        rk1 = jnp.sum(oh1 * (pre1 + cnt1_s[...]), axis=1, keepdims=True)
        rk2 = jnp.sum(oh2 * (pre2 + cnt2_s[...]), axis=1, keepdims=True)
        rk1_s[sl, :] = rk1
        rk2_s[sl, :] = rk2
        cnt1_s[...] = cnt1_s[...] + jnp.sum(oh1, axis=0, keepdims=True)
        cnt2_s[...] = cnt2_s[...] + jnp.sum(oh2, axis=0, keepdims=True)

    @pl.when(ph == 1)
    def _slots():
        @pl.when(c == 0)
        def _offsets():
            total = cnt1_s[...] + cnt2_s[...]
            padded = jnp.floor((total + (TILE_ - 1)) * (1.0 / TILE_)) * TILE_
            ea = lax.broadcasted_iota(jnp.int32, (NEXP_, NEXP_), 0)
            eb = lax.broadcasted_iota(jnp.int32, (NEXP_, NEXP_), 1)
            sut = (ea < eb).astype(jnp.float32)
            off = jnp.dot(padded, sut, preferred_element_type=jnp.float32)
            off1_s[...] = off
            off2_s[...] = off + cnt1_s[...]
            endv = off + padded
            ti = (lax.broadcasted_iota(jnp.int32, (NTILE_, NEXP_), 0)
                  * TILE_).astype(jnp.float32)
            tex = jnp.sum((ti >= endv).astype(jnp.float32), axis=1,
                          keepdims=True)
            te_ref[...] = tex.astype(jnp.int32)

        s1 = jnp.sum(oh1 * off1_s[...], axis=1, keepdims=True) + rk1_s[sl, :]
        s2 = jnp.sum(oh2 * off2_s[...], axis=1, keepdims=True) + rk2_s[sl, :]
        s1_ref[...] = s1.astype(jnp.int32)
        s2_ref[...] = s2.astype(jnp.int32)


def _dispatch_sc(x_hbm, s1_hbm, s2_hbm, xs_hbm, rows_v, i1_v, i2_v,
                 sem1, sem2):
    wid = lax.axis_index("s") * _NC + lax.axis_index("c")
    for ch in range(_TPW // _SCCH):
        base = wid * _TPW + ch * _SCCH
        pltpu.sync_copy(x_hbm.at[pl.ds(base, _SCCH)], rows_v)
        pltpu.sync_copy(s1_hbm.at[pl.ds(base, _SCCH)], i1_v)
        pltpu.sync_copy(s2_hbm.at[pl.ds(base, _SCCH)], i2_v)
        c1 = pltpu.async_copy(rows_v, xs_hbm.at[i1_v], sem1)
        c2 = pltpu.async_copy(rows_v, xs_hbm.at[i2_v], sem2)
        c1.wait()
        c2.wait()


def _ffn_body(te_ref, xs_ref, ew1_ref, eb1_ref, ew2_ref, eb2_ref, y_ref):
    @pl.when(te_ref[pl.program_id(0)] < NEXP_)
    def _compute():
        h = jnp.dot(xs_ref[...], ew1_ref[0],
                    preferred_element_type=jnp.float32)
        h = h + eb1_ref[0, 0, :]
        h = h * jax.nn.sigmoid(h)
        y = jnp.dot(h, ew2_ref[0], preferred_element_type=jnp.float32)
        y_ref[...] = y + eb2_ref[0, 0, :]


def _collect_sc(y_hbm, s1_hbm, s2_hbm, g1_hbm, g2_hbm, buf_v, i1_v, i2_v,
                sem):
    wid = lax.axis_index("s") * _NC + lax.axis_index("c")
    for ch in range(_TPW // _SCCH):
        base = wid * _TPW + ch * _SCCH
        pltpu.sync_copy(s1_hbm.at[pl.ds(base, _SCCH)], i1_v)
        pltpu.sync_copy(s2_hbm.at[pl.ds(base, _SCCH)], i2_v)
        pltpu.async_copy(y_hbm.at[i1_v], buf_v, sem).wait()
        pltpu.sync_copy(buf_v, g1_hbm.at[pl.ds(base, _SCCH)])
        pltpu.async_copy(y_hbm.at[i2_v], buf_v, sem).wait()
        pltpu.sync_copy(buf_v, g2_hbm.at[pl.ds(base, _SCCH)])


def _combine_body(g1_ref, g2_ref, p1_ref, p2_ref, out_ref):
    out_ref[...] = g1_ref[...] * p1_ref[...] + g2_ref[...] * p2_ref[...]


def kernel(input, rw1, rb1, rw2, rb2, ew1, eb1, ew2, eb2):
    bq, sq, d = input.shape
    x = input.reshape(bq * sq, d)

    e1, e2, p1, p2 = pl.pallas_call(
        _router_body,
        grid=(NCHUNK_,),
        in_specs=[
            pl.BlockSpec((CHUNK_, EMB_), lambda c: (c, 0)),
            pl.BlockSpec((EMB_, HID_), lambda c: (0, 0)),
            pl.BlockSpec((1, HID_), lambda c: (0, 0)),
            pl.BlockSpec((HID_, NEXP_), lambda c: (0, 0)),
            pl.BlockSpec((1, NEXP_), lambda c: (0, 0)),
        ],
        out_specs=[
            pl.BlockSpec((CHUNK_, 1), lambda c: (c, 0)),
            pl.BlockSpec((CHUNK_, 1), lambda c: (c, 0)),
            pl.BlockSpec((CHUNK_, 1), lambda c: (c, 0)),
            pl.BlockSpec((CHUNK_, 1), lambda c: (c, 0)),
        ],
        out_shape=[
            jax.ShapeDtypeStruct((NTOK_, 1), jnp.int32),
            jax.ShapeDtypeStruct((NTOK_, 1), jnp.int32),
            jax.ShapeDtypeStruct((NTOK_, 1), jnp.float32),
            jax.ShapeDtypeStruct((NTOK_, 1), jnp.float32),
        ],
    )(x, rw1, rb1.reshape(1, HID_), rw2, rb2.reshape(1, NEXP_))

    s1, s2, te = pl.pallas_call(
        _book_body,
        grid=(2, NCHUNK_),
        in_specs=[
            pl.BlockSpec((CHUNK_, 1), lambda p, c: (c, 0)),
            pl.BlockSpec((CHUNK_, 1), lambda p, c: (c, 0)),
        ],
        out_specs=[
            pl.BlockSpec((CHUNK_, 1), lambda p, c: (c, 0)),
            pl.BlockSpec((CHUNK_, 1), lambda p, c: (c, 0)),
            pl.BlockSpec((NTILE_, 1), lambda p, c: (0, 0)),
        ],
        out_shape=[
            jax.ShapeDtypeStruct((NTOK_, 1), jnp.int32),
            jax.ShapeDtypeStruct((NTOK_, 1), jnp.int32),
            jax.ShapeDtypeStruct((NTILE_, 1), jnp.int32),
        ],
        scratch_shapes=[
            pltpu.VMEM((1, NEXP_), jnp.float32),
            pltpu.VMEM((1, NEXP_), jnp.float32),
            pltpu.VMEM((NTOK_, 1), jnp.float32),
            pltpu.VMEM((NTOK_, 1), jnp.float32),
            pltpu.VMEM((1, NEXP_), jnp.float32),
            pltpu.VMEM((1, NEXP_), jnp.float32),
        ],
    )(e1, e2)

    s1f = s1.reshape(NTOK_)
    s2f = s2.reshape(NTOK_)
    tef = te.reshape(NTILE_)

    mesh = plsc.VectorSubcoreMesh(core_axis_name="c", subcore_axis_name="s")

    dispatch = pl.kernel(
        _dispatch_sc,
        mesh=mesh,
        out_type=jax.ShapeDtypeStruct((NSLOT_, EMB_), jnp.float32),
        scratch_types=[
            pltpu.VMEM((_SCCH, EMB_), jnp.float32),
            pltpu.VMEM((_SCCH,), jnp.int32),
            pltpu.VMEM((_SCCH,), jnp.int32),
            pltpu.SemaphoreType.DMA,
            pltpu.SemaphoreType.DMA,
        ],
    )
    xs = dispatch(x, s1f, s2f)

    y = pl.pallas_call(
        _ffn_body,
        grid_spec=pltpu.PrefetchScalarGridSpec(
            num_scalar_prefetch=1,
            grid=(NTILE_,),
            in_specs=[
                pl.BlockSpec((TILE_, EMB_), lambda i, te_r: (i, 0)),
                pl.BlockSpec((1, EMB_, HID_),
                             lambda i, te_r: (jnp.minimum(te_r[i], NEXP_ - 1),
                                              0, 0)),
                pl.BlockSpec((1, 1, HID_),
                             lambda i, te_r: (jnp.minimum(te_r[i], NEXP_ - 1),
                                              0, 0)),
                pl.BlockSpec((1, HID_, EMB_),
                             lambda i, te_r: (jnp.minimum(te_r[i], NEXP_ - 1),
                                              0, 0)),
                pl.BlockSpec((1, 1, EMB_),
                             lambda i, te_r: (jnp.minimum(te_r[i], NEXP_ - 1),
                                              0, 0)),
            ],
            out_specs=pl.BlockSpec((TILE_, EMB_), lambda i, te_r: (i, 0)),
        ),
        out_shape=jax.ShapeDtypeStruct((NSLOT_, EMB_), jnp.float32),
    )(tef, xs, ew1, eb1.reshape(NEXP_, 1, HID_), ew2,
      eb2.reshape(NEXP_, 1, EMB_))

    collect = pl.kernel(
        _collect_sc,
        mesh=mesh,
        out_type=[
            jax.ShapeDtypeStruct((NTOK_, EMB_), jnp.float32),
            jax.ShapeDtypeStruct((NTOK_, EMB_), jnp.float32),
        ],
        scratch_types=[
            pltpu.VMEM((_SCCH, EMB_), jnp.float32),
            pltpu.VMEM((_SCCH,), jnp.int32),
            pltpu.VMEM((_SCCH,), jnp.int32),
            pltpu.SemaphoreType.DMA,
        ],
    )
    g1, g2 = collect(y, s1f, s2f)

    out = pl.pallas_call(
        _combine_body,
        grid=(NCHUNK_,),
        in_specs=[
            pl.BlockSpec((CHUNK_, EMB_), lambda c: (c, 0)),
            pl.BlockSpec((CHUNK_, EMB_), lambda c: (c, 0)),
            pl.BlockSpec((CHUNK_, 1), lambda c: (c, 0)),
            pl.BlockSpec((CHUNK_, 1), lambda c: (c, 0)),
        ],
        out_specs=pl.BlockSpec((CHUNK_, EMB_), lambda c: (c, 0)),
        out_shape=jax.ShapeDtypeStruct((NTOK_, EMB_), jnp.float32),
    )(g1, g2, p1, p2)

    return out.reshape(bq, sq, d)
